# baseline (speedup 1.0000x reference)
import functools

import jax
import jax.numpy as jnp
from jax import lax
from jax.experimental import pallas as pl
from jax.experimental.pallas import tpu as pltpu
from jax.experimental.pallas import tpu_sc as plsc

_NUM_EMB = 1024
_DIM = 64
_BLK = 256
_N = 36864

_info = plsc.get_sparse_core_info()
_NC, _NS = _info.num_cores, _info.num_subcores
_NW = _NC * _NS
_B_PER_W = _N // _NW


def _prep_block(cb_ref, e2_ref, m2cb_ref, cbpad_ref):
    cb = cb_ref[...]
    e2_col = jnp.sum(cb * cb, axis=1, keepdims=True)
    e2_ref[...] = lax.transpose(e2_col, (1, 0))
    m2cb_ref[...] = cb * -2.0
    cbpad_ref[...] = jnp.concatenate([cb, jnp.zeros_like(cb)], axis=1)


def _vq_block(x_ref, e2_ref, m2cb_ref, enc_ref, idx_ref):
    x = x_ref[...]
    scores2 = lax.dot_general(
        x, m2cb_ref[...], (((1,), (1,)), ((), ())),
        preferred_element_type=jnp.float32,
    )
    x2 = jnp.sum(x * x, axis=1, keepdims=True)
    dist = (x2 + e2_ref[...]) + scores2
    min_d = jnp.min(dist, axis=1, keepdims=True)
    iota = lax.broadcasted_iota(jnp.int32, dist.shape, 1)
    cand = jnp.where(dist <= min_d, iota, jnp.int32(_NUM_EMB))
    idx = jnp.min(cand, axis=1, keepdims=True)
    enc_ref[...] = (iota == idx).astype(jnp.float32)
    idx_ref[...] = idx


@functools.partial(
    pl.kernel,
    mesh=plsc.VectorSubcoreMesh(core_axis_name="c", subcore_axis_name="s"),
    out_type=jax.ShapeDtypeStruct((_N, 2 * _DIM), jnp.float32),
    scratch_types=[
        pltpu.VMEM((_B_PER_W // 2,), jnp.int32),
        pltpu.VMEM((_B_PER_W // 2, 2 * _DIM), jnp.float32),
        pltpu.SemaphoreType.DMA,
    ],
)
def _sc_gather(idx_hbm, cbpad_hbm, out_hbm, idx_v, rows_v, sem):
    wid = lax.axis_index("s") * _NC + lax.axis_index("c")
    half = _B_PER_W // 2
    for h in range(2):
        base = wid * _B_PER_W + h * half
        pltpu.sync_copy(idx_hbm.at[pl.ds(base, half)], idx_v)
        pltpu.async_copy(cbpad_hbm.at[idx_v], rows_v, sem).wait()
        pltpu.sync_copy(rows_v, out_hbm.at[pl.ds(base, half)])


def kernel(inputs, codebook):
    input_shape = inputs.shape
    flat = inputs.reshape(-1, _DIM)
    n = flat.shape[0]
    grid = n // _BLK

    e2_row, m2cb, cbpad = pl.pallas_call(
        _prep_block,
        out_shape=[
            jax.ShapeDtypeStruct((1, _NUM_EMB), jnp.float32),
            jax.ShapeDtypeStruct((_NUM_EMB, _DIM), jnp.float32),
            jax.ShapeDtypeStruct((_NUM_EMB, 2 * _DIM), jnp.float32),
        ],
    )(codebook)

    enc, idx = pl.pallas_call(
        _vq_block,
        grid=(grid,),
        in_specs=[
            pl.BlockSpec((_BLK, _DIM), lambda i: (i, 0)),
            pl.BlockSpec((1, _NUM_EMB), lambda i: (0, 0)),
            pl.BlockSpec((_NUM_EMB, _DIM), lambda i: (0, 0)),
        ],
        out_specs=[
            pl.BlockSpec((_BLK, _NUM_EMB), lambda i: (i, 0)),
            pl.BlockSpec((_BLK, 1), lambda i: (i, 0)),
        ],
        out_shape=[
            jax.ShapeDtypeStruct((n, _NUM_EMB), jnp.float32),
            jax.ShapeDtypeStruct((n, 1), jnp.int32),
        ],
    )(flat, e2_row, m2cb)

    quant = _sc_gather(idx.reshape(-1), cbpad)[:, :_DIM]
    return quant.reshape(input_shape), enc


# speedup vs baseline: 1.8872x; 1.8872x over previous
import jax
import jax.numpy as jnp
from jax import lax
from jax.experimental import pallas as pl

_NUM_EMB = 1024
_DIM = 64
_BLK = 2304


def _prep_block(cb_ref, e2_ref, m2cb_ref):
    cb = cb_ref[...]
    e2_col = jnp.sum(cb * cb, axis=1, keepdims=True)
    e2_ref[...] = lax.transpose(e2_col, (1, 0))
    m2cb_ref[...] = cb * -2.0


def _vq_block(x_ref, cb_ref, e2_ref, m2cb_ref, enc_ref, q_ref):
    x = x_ref[...]
    scores2 = lax.dot_general(
        x, m2cb_ref[...], (((1,), (1,)), ((), ())),
        preferred_element_type=jnp.float32,
    )
    x2 = jnp.sum(x * x, axis=1, keepdims=True)
    dist = (x2 + e2_ref[...]) + scores2
    min_d = jnp.min(dist, axis=1, keepdims=True)
    enc = (dist <= min_d).astype(jnp.float32)
    enc_ref[...] = enc
    q_ref[...] = lax.dot_general(
        enc, cb_ref[...], (((1,), (0,)), ((), ())),
        preferred_element_type=jnp.float32,
    )


def kernel(inputs, codebook):
    input_shape = inputs.shape
    flat = inputs.reshape(-1, _DIM)
    n = flat.shape[0]
    grid = n // _BLK

    e2_row, m2cb = pl.pallas_call(
        _prep_block,
        out_shape=[
            jax.ShapeDtypeStruct((1, _NUM_EMB), jnp.float32),
            jax.ShapeDtypeStruct((_NUM_EMB, _DIM), jnp.float32),
        ],
    )(codebook)

    enc, quant = pl.pallas_call(
        _vq_block,
        grid=(grid,),
        in_specs=[
            pl.BlockSpec((_BLK, _DIM), lambda i: (i, 0)),
            pl.BlockSpec((_NUM_EMB, _DIM), lambda i: (0, 0)),
            pl.BlockSpec((1, _NUM_EMB), lambda i: (0, 0)),
            pl.BlockSpec((_NUM_EMB, _DIM), lambda i: (0, 0)),
        ],
        out_specs=[
            pl.BlockSpec((_BLK, _NUM_EMB), lambda i: (i, 0)),
            pl.BlockSpec((_BLK, _DIM), lambda i: (i, 0)),
        ],
        out_shape=[
            jax.ShapeDtypeStruct((n, _NUM_EMB), jnp.float32),
            jax.ShapeDtypeStruct((n, _DIM), jnp.float32),
        ],
    )(flat, codebook, e2_row, m2cb)

    return quant.reshape(input_shape), enc


# speedup vs baseline: 1.8879x; 1.0004x over previous
import jax
import jax.numpy as jnp
from jax import lax
from jax.experimental import pallas as pl

_NUM_EMB = 1024
_DIM = 64
_BLK = 3072


def _prep_block(cb_ref, e2_ref, m2cb_ref):
    cb = cb_ref[...]
    e2_col = jnp.sum(cb * cb, axis=1, keepdims=True)
    e2_ref[...] = lax.transpose(e2_col, (1, 0))
    m2cb_ref[...] = cb * -2.0


def _vq_block(x_ref, cb_ref, e2_ref, m2cb_ref, enc_ref, q_ref):
    x = x_ref[...]
    scores2 = lax.dot_general(
        x, m2cb_ref[...], (((1,), (1,)), ((), ())),
        preferred_element_type=jnp.float32,
    )
    x2 = jnp.sum(x * x, axis=1, keepdims=True)
    dist = (x2 + e2_ref[...]) + scores2
    min_d = jnp.min(dist, axis=1, keepdims=True)
    enc = (dist <= min_d).astype(jnp.float32)
    enc_ref[...] = enc
    q_ref[...] = lax.dot_general(
        enc, cb_ref[...], (((1,), (0,)), ((), ())),
        preferred_element_type=jnp.float32,
    )


def kernel(inputs, codebook):
    input_shape = inputs.shape
    flat = inputs.reshape(-1, _DIM)
    n = flat.shape[0]
    grid = n // _BLK

    e2_row, m2cb = pl.pallas_call(
        _prep_block,
        out_shape=[
            jax.ShapeDtypeStruct((1, _NUM_EMB), jnp.float32),
            jax.ShapeDtypeStruct((_NUM_EMB, _DIM), jnp.float32),
        ],
    )(codebook)

    enc, quant = pl.pallas_call(
        _vq_block,
        grid=(grid,),
        in_specs=[
            pl.BlockSpec((_BLK, _DIM), lambda i: (i, 0)),
            pl.BlockSpec((_NUM_EMB, _DIM), lambda i: (0, 0)),
            pl.BlockSpec((1, _NUM_EMB), lambda i: (0, 0)),
            pl.BlockSpec((_NUM_EMB, _DIM), lambda i: (0, 0)),
        ],
        out_specs=[
            pl.BlockSpec((_BLK, _NUM_EMB), lambda i: (i, 0)),
            pl.BlockSpec((_BLK, _DIM), lambda i: (i, 0)),
        ],
        out_shape=[
            jax.ShapeDtypeStruct((n, _NUM_EMB), jnp.float32),
            jax.ShapeDtypeStruct((n, _DIM), jnp.float32),
        ],
    )(flat, codebook, e2_row, m2cb)

    return quant.reshape(input_shape), enc


# speedup vs baseline: 1.8936x; 1.0030x over previous
import jax
import jax.numpy as jnp
from jax import lax
from jax.experimental import pallas as pl
from jax.experimental.pallas import tpu as pltpu

_NUM_EMB = 1024
_DIM = 64
_BLK = 3072


def _vq_block(x_ref, cb_ref, enc_ref, q_ref, e2_s, m2cb_s):
    @pl.when(pl.program_id(0) == 0)
    def _prep():
        cb = cb_ref[...]
        e2_col = jnp.sum(cb * cb, axis=1, keepdims=True)
        e2_s[...] = lax.transpose(e2_col, (1, 0))
        m2cb_s[...] = cb * -2.0

    x = x_ref[...]
    scores2 = lax.dot_general(
        x, m2cb_s[...], (((1,), (1,)), ((), ())),
        preferred_element_type=jnp.float32,
    )
    x2 = jnp.sum(x * x, axis=1, keepdims=True)
    dist = (x2 + e2_s[...]) + scores2
    min_d = jnp.min(dist, axis=1, keepdims=True)
    enc = (dist <= min_d).astype(jnp.float32)
    enc_ref[...] = enc
    q_ref[...] = lax.dot_general(
        enc, cb_ref[...], (((1,), (0,)), ((), ())),
        preferred_element_type=jnp.float32,
    )


def kernel(inputs, codebook):
    input_shape = inputs.shape
    flat = inputs.reshape(-1, _DIM)
    n = flat.shape[0]
    grid = n // _BLK

    enc, quant = pl.pallas_call(
        _vq_block,
        grid=(grid,),
        in_specs=[
            pl.BlockSpec((_BLK, _DIM), lambda i: (i, 0)),
            pl.BlockSpec((_NUM_EMB, _DIM), lambda i: (0, 0)),
        ],
        out_specs=[
            pl.BlockSpec((_BLK, _NUM_EMB), lambda i: (i, 0)),
            pl.BlockSpec((_BLK, _DIM), lambda i: (i, 0)),
        ],
        out_shape=[
            jax.ShapeDtypeStruct((n, _NUM_EMB), jnp.float32),
            jax.ShapeDtypeStruct((n, _DIM), jnp.float32),
        ],
        scratch_shapes=[
            pltpu.VMEM((1, _NUM_EMB), jnp.float32),
            pltpu.VMEM((_NUM_EMB, _DIM), jnp.float32),
        ],
    )(flat, codebook)

    return quant.reshape(input_shape), enc


# speedup vs baseline: 2.0002x; 1.0563x over previous
import jax
import jax.numpy as jnp
from jax import lax
from jax.experimental import pallas as pl
from jax.experimental.pallas import tpu as pltpu

_NUM_EMB = 1024
_DIM = 64
_IMGS = 4


def _vq_block(x_ref, cb_ref, enc_ref, q_ref, e2_s, m2cb_s):
    @pl.when(pl.program_id(0) == 0)
    def _prep():
        cb = cb_ref[...]
        e2_col = jnp.sum(cb * cb, axis=1, keepdims=True)
        e2_s[...] = lax.transpose(e2_col, (1, 0))
        m2cb_s[...] = cb * -2.0

    x3 = x_ref[...]
    x = x3.reshape(-1, _DIM)
    scores2 = lax.dot_general(
        x, m2cb_s[...], (((1,), (1,)), ((), ())),
        preferred_element_type=jnp.float32,
    )
    x2 = jnp.sum(x * x, axis=1, keepdims=True)
    dist = (x2 + e2_s[...]) + scores2
    min_d = jnp.min(dist, axis=1, keepdims=True)
    enc = (dist <= min_d).astype(jnp.float32)
    enc_ref[...] = enc
    q = lax.dot_general(
        enc, cb_ref[...], (((1,), (0,)), ((), ())),
        preferred_element_type=jnp.float32,
    )
    q_ref[...] = q.reshape(x3.shape)


def kernel(inputs, codebook):
    nimg, npos, _ = inputs.shape
    n = nimg * npos
    grid = nimg // _IMGS

    enc, quant = pl.pallas_call(
        _vq_block,
        grid=(grid,),
        in_specs=[
            pl.BlockSpec((_IMGS, npos, _DIM), lambda i: (i, 0, 0)),
            pl.BlockSpec((_NUM_EMB, _DIM), lambda i: (0, 0)),
        ],
        out_specs=[
            pl.BlockSpec((_IMGS * npos, _NUM_EMB), lambda i: (i, 0)),
            pl.BlockSpec((_IMGS, npos, _DIM), lambda i: (i, 0, 0)),
        ],
        out_shape=[
            jax.ShapeDtypeStruct((n, _NUM_EMB), jnp.float32),
            jax.ShapeDtypeStruct(inputs.shape, jnp.float32),
        ],
        scratch_shapes=[
            pltpu.VMEM((1, _NUM_EMB), jnp.float32),
            pltpu.VMEM((_NUM_EMB, _DIM), jnp.float32),
        ],
    )(inputs, codebook)

    return quant, enc


# speedup vs baseline: 2.0043x; 1.0020x over previous
import jax
import jax.numpy as jnp
from jax import lax
from jax.experimental import pallas as pl
from jax.experimental.pallas import tpu as pltpu

_NUM_EMB = 1024
_DIM = 64
_IMGS = 4


def _vq_block(x_ref, cb_ref, enc_ref, q_ref):
    cb = cb_ref[...]
    e2_col = jnp.sum(cb * cb, axis=1, keepdims=True)
    e2_row = lax.transpose(e2_col, (1, 0))

    x3 = x_ref[...]
    x = x3.reshape(-1, _DIM)
    scores2 = lax.dot_general(
        x, cb * -2.0, (((1,), (1,)), ((), ())),
        preferred_element_type=jnp.float32,
    )
    x2 = jnp.sum(x * x, axis=1, keepdims=True)
    dist = (x2 + e2_row) + scores2
    min_d = jnp.min(dist, axis=1, keepdims=True)
    enc = (dist <= min_d).astype(jnp.float32)
    enc_ref[...] = enc
    q = lax.dot_general(
        enc, cb, (((1,), (0,)), ((), ())),
        preferred_element_type=jnp.float32,
    )
    q_ref[...] = q.reshape(x3.shape)


def kernel(inputs, codebook):
    nimg, npos, _ = inputs.shape
    n = nimg * npos
    grid = nimg // _IMGS

    enc, quant = pl.pallas_call(
        _vq_block,
        grid=(grid,),
        in_specs=[
            pl.BlockSpec((_IMGS, npos, _DIM), lambda i: (i, 0, 0)),
            pl.BlockSpec((_NUM_EMB, _DIM), lambda i: (0, 0)),
        ],
        out_specs=[
            pl.BlockSpec((_IMGS * npos, _NUM_EMB), lambda i: (i, 0)),
            pl.BlockSpec((_IMGS, npos, _DIM), lambda i: (i, 0, 0)),
        ],
        out_shape=[
            jax.ShapeDtypeStruct((n, _NUM_EMB), jnp.float32),
            jax.ShapeDtypeStruct(inputs.shape, jnp.float32),
        ],
        compiler_params=pltpu.CompilerParams(
            dimension_semantics=("parallel",),
        ),
    )(inputs, codebook)

    return quant, enc
